# Initial kernel scaffold; baseline (speedup 1.0000x reference)
import jax
import jax.numpy as jnp
from jax.experimental import pallas as pl


def kernel(x, edge_index, W_self_0, W_neigh_0, b_0, W_self_1, W_neigh_1, b_1, W_self_2, W_neigh_2, b_2, gamma_0, beta_0, gamma_1, beta_1):
    raise NotImplementedError("write your pallas kernel here")


# speedup vs baseline: 7.5130x; 7.5130x over previous
import functools

import jax
import jax.numpy as jnp
from jax import lax
from jax.experimental import pallas as pl
from jax.experimental.pallas import tpu as pltpu
from jax.experimental.pallas import tpu_sc as plsc

N = 10000
E = 320000
D = 128
H = 64
C = 80
EPS = 1e-5

N_PAD = 10240
NW = 32
CHUNK = 128
CH = 80
E_PAD = NW * CH * CHUNK
CHP = 88
ROWS_PER_TILE = N_PAD // 16


def _make_sc_agg():
    mesh = plsc.VectorSubcoreMesh(core_axis_name="c", subcore_axis_name="s")

    @functools.partial(
        pl.kernel,
        out_type=jax.ShapeDtypeStruct((2 * N_PAD, C), jnp.float32),
        mesh=mesh,
        scratch_types=[
            pltpu.VMEM((CHP, CHUNK), jnp.int32),
            pltpu.VMEM((CH, CHUNK), jnp.int32),
            pltpu.VMEM((2, CHUNK, C), jnp.float32),
            pltpu.VMEM((128, C), jnp.float32),
            pltpu.VMEM_SHARED((N_PAD, C), jnp.float32),
            pltpu.SemaphoreType.DMA,
            pltpu.SemaphoreType.DMA,
        ],
        compiler_params=pltpu.CompilerParams(use_tc_tiling_on_sc=False),
    )
    def agg(table_hbm, src_hbm, dst_hbm, out_hbm,
            src_v, dst_v, rows_v, stage_v, acc_sh, sem0, sem1):
        cid = lax.axis_index("c")
        sid = lax.axis_index("s")
        w = cid * 16 + sid

        pltpu.sync_copy(src_hbm.at[pl.ds(w * CHP, CHP)], src_v)
        pltpu.sync_copy(dst_hbm.at[pl.ds(w * CH, CH)], dst_v)

        zv = jnp.zeros((16,), jnp.float32)

        def zrow(i, carry):
            for j in range(C // 16):
                stage_v[i, pl.ds(j * 16, 16)] = zv
            return carry

        lax.fori_loop(0, 128, zrow, 0)
        for k in range(ROWS_PER_TILE // 128):
            pltpu.sync_copy(
                stage_v, acc_sh.at[pl.ds(sid * ROWS_PER_TILE + k * 128, 128)])
        plsc.subcore_barrier()

        sems = (sem0, sem1)
        pltpu.async_copy(table_hbm.at[src_v.at[0]], rows_v.at[0], sem0)
        pltpu.async_copy(table_hbm.at[src_v.at[1]], rows_v.at[1], sem1)

        def body(jh, carry):
            for b in range(2):
                jj = jh * 2 + b
                pltpu.make_async_copy(
                    table_hbm.at[src_v.at[jj]], rows_v.at[b], sems[b]).wait()
                pltpu.sync_copy(rows_v.at[b], acc_sh.at[dst_v.at[jj]], add=True)
                pltpu.async_copy(
                    table_hbm.at[src_v.at[jj + 2]], rows_v.at[b], sems[b])
            return carry

        lax.fori_loop(0, CH // 2, body, 0)
        pltpu.make_async_copy(
            table_hbm.at[src_v.at[CH]], rows_v.at[0], sem0).wait()
        pltpu.make_async_copy(
            table_hbm.at[src_v.at[CH + 1]], rows_v.at[1], sem1).wait()
        plsc.subcore_barrier()

        for k in range(ROWS_PER_TILE // 128):
            r0 = sid * ROWS_PER_TILE + k * 128
            pltpu.sync_copy(acc_sh.at[pl.ds(r0, 128)], stage_v)
            pltpu.sync_copy(stage_v, out_hbm.at[pl.ds(cid * N_PAD + r0, 128)])

    return agg


_sc_agg = _make_sc_agg()


_DOT = functools.partial(
    jnp.dot, preferred_element_type=jnp.float32, precision=lax.Precision.HIGHEST)

BLK = 1000
GRID = N // BLK


def _neigh(aa0_ref, aa1_ref, ab0_ref, ab1_ref, rdeg, wn_ref):
    hna = aa0_ref[0, :, 0:H] + aa1_ref[0, :, 0:H]
    hnb = ab0_ref[0, :, 0:H] + ab1_ref[0, :, 0:H]
    return _DOT(hna * rdeg, wn_ref[0:H, :]) + _DOT(hnb * rdeg, wn_ref[H:D, :])


def _accum_stats(z, z_ref, s1_ref, s2_ref):
    z_ref[...] = z

    @pl.when(pl.program_id(0) == 0)
    def _():
        s1_ref[...] = jnp.zeros_like(s1_ref)
        s2_ref[...] = jnp.zeros_like(s2_ref)

    s1_ref[...] += jnp.sum(z, axis=0, keepdims=True)
    s2_ref[...] += jnp.sum(z * z, axis=0, keepdims=True)


def _stats0_body(x_ref, aa0_ref, aa1_ref, ab0_ref, ab1_ref, ws_ref, wn_ref,
                 b_ref, z_ref, s1_ref, s2_ref, rdeg_ref):
    deg = jnp.sum(aa0_ref[0, :, H:C] + aa1_ref[0, :, H:C],
                  axis=1, keepdims=True)
    rdeg = 1.0 / jnp.maximum(deg, 1.0)
    rdeg_ref[...] = rdeg
    z = (_DOT(x_ref[...], ws_ref[...])
         + _neigh(aa0_ref, aa1_ref, ab0_ref, ab1_ref, rdeg, wn_ref)
         + b_ref[...])
    _accum_stats(z, z_ref, s1_ref, s2_ref)


def _stats_mid_body(ha_ref, hb_ref, aa0_ref, aa1_ref, ab0_ref, ab1_ref,
                    ws_ref, wn_ref, b_ref, z_ref, s1_ref, s2_ref, rdeg_ref):
    rdeg = ha_ref[:, H:H + 1]
    rdeg_ref[...] = rdeg
    z = (_DOT(ha_ref[:, 0:H], ws_ref[0:H, :])
         + _DOT(hb_ref[:, 0:H], ws_ref[H:D, :])
         + _neigh(aa0_ref, aa1_ref, ab0_ref, ab1_ref, rdeg, wn_ref)
         + b_ref[...])
    _accum_stats(z, z_ref, s1_ref, s2_ref)


def _apply_body(z_ref, s1_ref, s2_ref, g_ref, be_ref, rdeg_ref,
                outa_ref, outb_ref):
    m = s1_ref[...] * (1.0 / N)
    v = s2_ref[...] * (1.0 / N) - m * m
    scale = g_ref[...] * lax.rsqrt(v + EPS)
    h = jnp.maximum(scale * (z_ref[...] - m) + be_ref[...], 0.0)
    rdeg = rdeg_ref[...]
    outa_ref[...] = jnp.concatenate(
        [h[:, 0:H], rdeg, jnp.zeros((BLK, C - H - 1), jnp.float32)], axis=1)
    outb_ref[...] = jnp.concatenate(
        [h[:, H:D], jnp.zeros((BLK, C - H), jnp.float32)], axis=1)


def _last_body(ha_ref, hb_ref, aa0_ref, aa1_ref, ab0_ref, ab1_ref, ws_ref,
               wn_ref, b_ref, out_ref):
    rdeg = ha_ref[:, H:H + 1]
    out_ref[...] = (_DOT(ha_ref[:, 0:H], ws_ref[0:H, :])
                    + _DOT(hb_ref[:, 0:H], ws_ref[H:D, :])
                    + _neigh(aa0_ref, aa1_ref, ab0_ref, ab1_ref, rdeg, wn_ref)
                    + b_ref[...])


def _blk(i):
    return (i, 0)


def _fix(i):
    return (0, 0)


_HBLK = pl.BlockSpec((BLK, C), _blk)
_ROWBLK = pl.BlockSpec((BLK, D), _blk)
_RDEGBLK = pl.BlockSpec((BLK, 1), _blk)
_W_SPEC = pl.BlockSpec((D, D), _fix)
_ROW1 = pl.BlockSpec((1, D), _fix)
_AGG0 = pl.BlockSpec((1, BLK, C), lambda i: (0, i, 0))
_AGG1 = pl.BlockSpec((1, BLK, C), lambda i: (1, i, 0))

_Z_OUT = jax.ShapeDtypeStruct((N, D), jnp.float32)
_S_OUT = jax.ShapeDtypeStruct((1, D), jnp.float32)
_RDEG_OUT = jax.ShapeDtypeStruct((N, 1), jnp.float32)
_H_OUT = jax.ShapeDtypeStruct((N, C), jnp.float32)

_STATS_OUT_SHAPES = (_Z_OUT, _S_OUT, _S_OUT, _RDEG_OUT)
_STATS_OUT_SPECS = (_ROWBLK, pl.BlockSpec((1, D), _fix),
                    pl.BlockSpec((1, D), _fix), _RDEGBLK)

_stats0 = pl.pallas_call(
    _stats0_body,
    grid=(GRID,),
    in_specs=[_ROWBLK, _AGG0, _AGG1, _AGG0, _AGG1, _W_SPEC, _W_SPEC, _ROW1],
    out_specs=_STATS_OUT_SPECS,
    out_shape=_STATS_OUT_SHAPES)

_stats_mid = pl.pallas_call(
    _stats_mid_body,
    grid=(GRID,),
    in_specs=[_HBLK, _HBLK, _AGG0, _AGG1, _AGG0, _AGG1, _W_SPEC, _W_SPEC,
              _ROW1],
    out_specs=_STATS_OUT_SPECS,
    out_shape=_STATS_OUT_SHAPES)

_apply = pl.pallas_call(
    _apply_body,
    grid=(GRID,),
    in_specs=[_ROWBLK, pl.BlockSpec((1, D), _fix), pl.BlockSpec((1, D), _fix),
              _ROW1, _ROW1, _RDEGBLK],
    out_specs=(_HBLK, _HBLK),
    out_shape=(_H_OUT, _H_OUT))

_last = pl.pallas_call(
    _last_body,
    grid=(GRID,),
    in_specs=[_HBLK, _HBLK, _AGG0, _AGG1, _AGG0, _AGG1, _W_SPEC, _W_SPEC,
              _ROW1],
    out_specs=_ROWBLK,
    out_shape=_Z_OUT)


def kernel(x, edge_index, W_self_0, W_neigh_0, b_0, W_self_1, W_neigh_1, b_1,
           W_self_2, W_neigh_2, b_2, gamma_0, beta_0, gamma_1, beta_1):
    src = edge_index[0]
    dst = edge_index[1]

    pad = E_PAD - E
    ar = jnp.arange(pad, dtype=jnp.int32)
    src_p = jnp.concatenate([src, ar % N]).reshape(NW, CH, CHUNK)
    dst_p = jnp.concatenate([dst, N + ar % (N_PAD - N)])
    extra = (jnp.arange(NW * (CHP - CH) * CHUNK, dtype=jnp.int32) * 37) % N
    src_full = jnp.concatenate(
        [src_p, extra.reshape(NW, CHP - CH, CHUNK)], axis=1).reshape(
            NW * CHP, CHUNK)
    dst_full = dst_p.reshape(NW * CH, CHUNK)

    xa = jnp.concatenate(
        [x[:, 0:H], jnp.ones((N, 1), jnp.float32),
         jnp.zeros((N, C - H - 1), jnp.float32)], axis=1)
    xb = jnp.concatenate(
        [x[:, H:D], jnp.zeros((N, C - H), jnp.float32)], axis=1)

    b0 = b_0.reshape(1, D)
    b1 = b_1.reshape(1, D)
    b2 = b_2.reshape(1, D)
    g0 = gamma_0.reshape(1, D)
    g1 = gamma_1.reshape(1, D)
    be0 = beta_0.reshape(1, D)
    be1 = beta_1.reshape(1, D)

    agg0a = _sc_agg(xa, src_full, dst_full).reshape(2, N_PAD, C)
    agg0b = _sc_agg(xb, src_full, dst_full).reshape(2, N_PAD, C)
    z0, s1_0, s2_0, rdeg = _stats0(x, agg0a, agg0a, agg0b, agg0b,
                                   W_self_0, W_neigh_0, b0)
    h1a, h1b = _apply(z0, s1_0, s2_0, g0, be0, rdeg)
    agg1a = _sc_agg(h1a, src_full, dst_full).reshape(2, N_PAD, C)
    agg1b = _sc_agg(h1b, src_full, dst_full).reshape(2, N_PAD, C)
    z1, s1_1, s2_1, rdeg1 = _stats_mid(h1a, h1b, agg1a, agg1a, agg1b, agg1b,
                                       W_self_1, W_neigh_1, b1)
    h2a, h2b = _apply(z1, s1_1, s2_1, g1, be1, rdeg1)
    agg2a = _sc_agg(h2a, src_full, dst_full).reshape(2, N_PAD, C)
    agg2b = _sc_agg(h2b, src_full, dst_full).reshape(2, N_PAD, C)
    return _last(h2a, h2b, agg2a, agg2a, agg2b, agg2b, W_self_2, W_neigh_2, b2)


# speedup vs baseline: 8.2560x; 1.0989x over previous
import functools

import jax
import jax.numpy as jnp
from jax import lax
from jax.experimental import pallas as pl
from jax.experimental.pallas import tpu as pltpu
from jax.experimental.pallas import tpu_sc as plsc

N = 10000
E = 320000
D = 128
H = 64
C = 80
EPS = 1e-5

N_PAD = 10240
NW = 32
CHUNK = 128
CH = 80
E_PAD = NW * CH * CHUNK
CHP = 88
ROWS_PER_TILE = N_PAD // 16


def _make_sc_agg():
    mesh = plsc.VectorSubcoreMesh(core_axis_name="c", subcore_axis_name="s")

    @functools.partial(
        pl.kernel,
        out_type=jax.ShapeDtypeStruct((2 * N_PAD, C), jnp.float32),
        mesh=mesh,
        scratch_types=[
            pltpu.VMEM((CHP, CHUNK), jnp.int32),
            pltpu.VMEM((CH, CHUNK), jnp.int32),
            pltpu.VMEM((2, CHUNK, C), jnp.float32),
            pltpu.VMEM((128, C), jnp.float32),
            pltpu.VMEM_SHARED((N_PAD, C), jnp.float32),
            pltpu.SemaphoreType.DMA,
            pltpu.SemaphoreType.DMA,
        ],
        compiler_params=pltpu.CompilerParams(use_tc_tiling_on_sc=False),
    )
    def agg(table_hbm, src_hbm, dst_hbm, out_hbm,
            src_v, dst_v, rows_v, stage_v, acc_sh, sem0, sem1):
        cid = lax.axis_index("c")
        sid = lax.axis_index("s")
        w = cid * 16 + sid

        pltpu.sync_copy(src_hbm.at[pl.ds(w * CHP, CHP)], src_v)
        pltpu.sync_copy(dst_hbm.at[pl.ds(w * CH, CH)], dst_v)

        zv = jnp.zeros((16,), jnp.float32)

        def zrow(i, carry):
            for j in range(C // 16):
                stage_v[i, pl.ds(j * 16, 16)] = zv
            return carry

        lax.fori_loop(0, 128, zrow, 0)
        for k in range(ROWS_PER_TILE // 128):
            pltpu.sync_copy(
                stage_v, acc_sh.at[pl.ds(sid * ROWS_PER_TILE + k * 128, 128)])
        plsc.subcore_barrier()

        sems = (sem0, sem1)
        pltpu.async_copy(table_hbm.at[src_v.at[0]], rows_v.at[0], sem0)
        pltpu.async_copy(table_hbm.at[src_v.at[1]], rows_v.at[1], sem1)

        def body(jh, carry):
            for b in range(2):
                jj = jh * 2 + b
                pltpu.make_async_copy(
                    table_hbm.at[src_v.at[jj]], rows_v.at[b], sems[b]).wait()
                pltpu.sync_copy(rows_v.at[b], acc_sh.at[dst_v.at[jj]], add=True)
                pltpu.async_copy(
                    table_hbm.at[src_v.at[jj + 2]], rows_v.at[b], sems[b])
            return carry

        lax.fori_loop(0, CH // 2, body, 0)
        pltpu.make_async_copy(
            table_hbm.at[src_v.at[CH]], rows_v.at[0], sem0).wait()
        pltpu.make_async_copy(
            table_hbm.at[src_v.at[CH + 1]], rows_v.at[1], sem1).wait()
        plsc.subcore_barrier()

        for k in range(ROWS_PER_TILE // 128):
            r0 = sid * ROWS_PER_TILE + k * 128
            pltpu.sync_copy(acc_sh.at[pl.ds(r0, 128)], stage_v)
            pltpu.sync_copy(stage_v, out_hbm.at[pl.ds(cid * N_PAD + r0, 128)])

    return agg


_sc_agg = _make_sc_agg()


_DOT = functools.partial(
    jnp.dot, preferred_element_type=jnp.float32, precision=lax.Precision.DEFAULT)

BLK = 1024
GRID = N_PAD // BLK


def _neigh(aa0_ref, aa1_ref, ab0_ref, ab1_ref, rdeg, wn_ref):
    hna = aa0_ref[:, 0:H] + aa1_ref[:, 0:H]
    hnb = ab0_ref[:, 0:H] + ab1_ref[:, 0:H]
    return _DOT(hna * rdeg, wn_ref[0:H, :]) + _DOT(hnb * rdeg, wn_ref[H:D, :])


def _accum_stats(z, z_ref, s1_ref, s2_ref):
    z_ref[...] = z

    @pl.when(pl.program_id(0) == 0)
    def _():
        s1_ref[...] = jnp.zeros_like(s1_ref)
        s2_ref[...] = jnp.zeros_like(s2_ref)

    row = (pl.program_id(0) * BLK
           + lax.broadcasted_iota(jnp.int32, (BLK, 1), 0))
    zm = jnp.where(row < N, z, 0.0)
    s1_ref[...] += jnp.sum(zm, axis=0, keepdims=True)
    s2_ref[...] += jnp.sum(zm * zm, axis=0, keepdims=True)


def _stats0_body(x_ref, aa0_ref, aa1_ref, ab0_ref, ab1_ref, ws_ref, wn_ref,
                 b_ref, z_ref, s1_ref, s2_ref, rdeg_ref):
    deg = jnp.sum(aa0_ref[:, H:C] + aa1_ref[:, H:C],
                  axis=1, keepdims=True)
    rdeg = 1.0 / jnp.maximum(deg, 1.0)
    rdeg_ref[...] = rdeg
    z = (_DOT(x_ref[...], ws_ref[...])
         + _neigh(aa0_ref, aa1_ref, ab0_ref, ab1_ref, rdeg, wn_ref)
         + b_ref[...])
    _accum_stats(z, z_ref, s1_ref, s2_ref)


def _stats_mid_body(ha_ref, hb_ref, aa0_ref, aa1_ref, ab0_ref, ab1_ref,
                    ws_ref, wn_ref, b_ref, z_ref, s1_ref, s2_ref, rdeg_ref):
    rdeg = ha_ref[:, H:H + 1]
    rdeg_ref[...] = rdeg
    z = (_DOT(ha_ref[:, 0:H], ws_ref[0:H, :])
         + _DOT(hb_ref[:, 0:H], ws_ref[H:D, :])
         + _neigh(aa0_ref, aa1_ref, ab0_ref, ab1_ref, rdeg, wn_ref)
         + b_ref[...])
    _accum_stats(z, z_ref, s1_ref, s2_ref)


def _apply_body(z_ref, s1_ref, s2_ref, g_ref, be_ref, rdeg_ref,
                outa_ref, outb_ref):
    m = s1_ref[...] * (1.0 / N)
    v = s2_ref[...] * (1.0 / N) - m * m
    scale = g_ref[...] * lax.rsqrt(v + EPS)
    h = jnp.maximum(scale * (z_ref[...] - m) + be_ref[...], 0.0)
    rdeg = rdeg_ref[...]
    outa_ref[...] = jnp.concatenate(
        [h[:, 0:H], rdeg, jnp.zeros((BLK, C - H - 1), jnp.float32)], axis=1)
    outb_ref[...] = jnp.concatenate(
        [h[:, H:D], jnp.zeros((BLK, C - H), jnp.float32)], axis=1)


def _blk(i):
    return (i, 0)


def _blk1(i):
    return (i + GRID, 0)


def _fix(i):
    return (0, 0)


def _last_body(ha_ref, hb_ref, aa0_ref, aa1_ref, ab0_ref, ab1_ref, ws_ref,
               wn_ref, b_ref, out_ref):
    rdeg = ha_ref[:, H:H + 1]
    out_ref[...] = (_DOT(ha_ref[:, 0:H], ws_ref[0:H, :])
                    + _DOT(hb_ref[:, 0:H], ws_ref[H:D, :])
                    + _neigh(aa0_ref, aa1_ref, ab0_ref, ab1_ref, rdeg, wn_ref)
                    + b_ref[...])


_HBLK = pl.BlockSpec((BLK, C), _blk)
_ROWBLK = pl.BlockSpec((BLK, D), _blk)
_RDEGBLK = pl.BlockSpec((BLK, 1), _blk)
_W_SPEC = pl.BlockSpec((D, D), _fix)
_ROW1 = pl.BlockSpec((1, D), _fix)
_AGG0 = pl.BlockSpec((BLK, C), _blk)
_AGG1 = pl.BlockSpec((BLK, C), _blk1)

_Z_OUT = jax.ShapeDtypeStruct((N_PAD, D), jnp.float32)
_S_OUT = jax.ShapeDtypeStruct((1, D), jnp.float32)
_RDEG_OUT = jax.ShapeDtypeStruct((N_PAD, 1), jnp.float32)
_H_OUT = jax.ShapeDtypeStruct((N_PAD, C), jnp.float32)
_FINAL_OUT = jax.ShapeDtypeStruct((N, D), jnp.float32)

_STATS_OUT_SHAPES = (_Z_OUT, _S_OUT, _S_OUT, _RDEG_OUT)
_STATS_OUT_SPECS = (_ROWBLK, pl.BlockSpec((1, D), _fix),
                    pl.BlockSpec((1, D), _fix), _RDEGBLK)

_stats0 = pl.pallas_call(
    _stats0_body,
    grid=(GRID,),
    in_specs=[_ROWBLK, _AGG0, _AGG1, _AGG0, _AGG1, _W_SPEC, _W_SPEC, _ROW1],
    out_specs=_STATS_OUT_SPECS,
    out_shape=_STATS_OUT_SHAPES)

_stats_mid = pl.pallas_call(
    _stats_mid_body,
    grid=(GRID,),
    in_specs=[_HBLK, _HBLK, _AGG0, _AGG1, _AGG0, _AGG1, _W_SPEC, _W_SPEC,
              _ROW1],
    out_specs=_STATS_OUT_SPECS,
    out_shape=_STATS_OUT_SHAPES)

_apply = pl.pallas_call(
    _apply_body,
    grid=(GRID,),
    in_specs=[_ROWBLK, pl.BlockSpec((1, D), _fix), pl.BlockSpec((1, D), _fix),
              _ROW1, _ROW1, _RDEGBLK],
    out_specs=(_HBLK, _HBLK),
    out_shape=(_H_OUT, _H_OUT))

_last = pl.pallas_call(
    _last_body,
    grid=(GRID,),
    in_specs=[_HBLK, _HBLK, _AGG0, _AGG1, _AGG0, _AGG1, _W_SPEC, _W_SPEC,
              _ROW1],
    out_specs=_ROWBLK,
    out_shape=_FINAL_OUT)


def kernel(x, edge_index, W_self_0, W_neigh_0, b_0, W_self_1, W_neigh_1, b_1,
           W_self_2, W_neigh_2, b_2, gamma_0, beta_0, gamma_1, beta_1):
    src = edge_index[0]
    dst = edge_index[1]

    pad = E_PAD - E
    ar = jnp.arange(pad, dtype=jnp.int32)
    src_p = jnp.concatenate([src, ar % N]).reshape(NW, CH, CHUNK)
    dst_p = jnp.concatenate([dst, N + ar % (N_PAD - N)])
    extra = (jnp.arange(NW * (CHP - CH) * CHUNK, dtype=jnp.int32) * 37) % N
    src_full = jnp.concatenate(
        [src_p, extra.reshape(NW, CHP - CH, CHUNK)], axis=1).reshape(
            NW * CHP, CHUNK)
    dst_full = dst_p.reshape(NW * CH, CHUNK)

    xa = jnp.pad(jnp.concatenate(
        [x[:, 0:H], jnp.ones((N, 1), jnp.float32),
         jnp.zeros((N, C - H - 1), jnp.float32)], axis=1),
        ((0, N_PAD - N), (0, 0)))
    xb = jnp.pad(jnp.concatenate(
        [x[:, H:D], jnp.zeros((N, C - H), jnp.float32)], axis=1),
        ((0, N_PAD - N), (0, 0)))

    b0 = b_0.reshape(1, D)
    b1 = b_1.reshape(1, D)
    b2 = b_2.reshape(1, D)
    g0 = gamma_0.reshape(1, D)
    g1 = gamma_1.reshape(1, D)
    be0 = beta_0.reshape(1, D)
    be1 = beta_1.reshape(1, D)

    agg0a = _sc_agg(xa, src_full, dst_full)
    agg0b = _sc_agg(xb, src_full, dst_full)
    z0, s1_0, s2_0, rdeg = _stats0(x, agg0a, agg0a, agg0b, agg0b,
                                   W_self_0, W_neigh_0, b0)
    h1a, h1b = _apply(z0, s1_0, s2_0, g0, be0, rdeg)
    agg1a = _sc_agg(h1a, src_full, dst_full)
    agg1b = _sc_agg(h1b, src_full, dst_full)
    z1, s1_1, s2_1, rdeg1 = _stats_mid(h1a, h1b, agg1a, agg1a, agg1b, agg1b,
                                       W_self_1, W_neigh_1, b1)
    h2a, h2b = _apply(z1, s1_1, s2_1, g1, be1, rdeg1)
    agg2a = _sc_agg(h2a, src_full, dst_full)
    agg2b = _sc_agg(h2b, src_full, dst_full)
    return _last(h2a, h2b, agg2a, agg2a, agg2b, agg2b, W_self_2, W_neigh_2, b2)


# speedup vs baseline: 8.7678x; 1.0620x over previous
import functools

import jax
import jax.numpy as jnp
from jax import lax
from jax.experimental import pallas as pl
from jax.experimental.pallas import tpu as pltpu
from jax.experimental.pallas import tpu_sc as plsc

N = 10000
E = 320000
D = 128
H = 64
C = 80
EPS = 1e-5

N_PAD = 10240
NT = 16
CHUNK = 128
CH = 160
E_PAD = NT * CH * CHUNK
CHP = 168
ROWS_PER_TILE = N_PAD // NT


def _make_sc_agg():
    mesh = plsc.VectorSubcoreMesh(core_axis_name="c", subcore_axis_name="s")

    @functools.partial(
        pl.kernel,
        out_type=jax.ShapeDtypeStruct((2 * N_PAD, C), jnp.float32),
        mesh=mesh,
        scratch_types=[
            pltpu.VMEM((CHP, CHUNK), jnp.int32),
            pltpu.VMEM((CH, CHUNK), jnp.int32),
            pltpu.VMEM((2, CHUNK, C), jnp.float32),
            pltpu.VMEM((128, C), jnp.float32),
            pltpu.VMEM_SHARED((N_PAD, C), jnp.float32),
            pltpu.SemaphoreType.DMA,
            pltpu.SemaphoreType.DMA,
        ],
        compiler_params=pltpu.CompilerParams(use_tc_tiling_on_sc=False),
    )
    def agg(ta_hbm, tb_hbm, src_hbm, dst_hbm, out_hbm,
            src_v, dst_v, rows_v, stage_v, acc_sh, sem0, sem1):
        cid = lax.axis_index("c")
        sid = lax.axis_index("s")

        pltpu.sync_copy(src_hbm.at[pl.ds(sid * CHP, CHP)], src_v)
        pltpu.sync_copy(dst_hbm.at[pl.ds(sid * CH, CH)], dst_v)

        zv = jnp.zeros((16,), jnp.float32)

        def zrow(i, carry):
            for j in range(C // 16):
                stage_v[i, pl.ds(j * 16, 16)] = zv
            return carry

        lax.fori_loop(0, 128, zrow, 0)
        for k in range(ROWS_PER_TILE // 128):
            pltpu.sync_copy(
                stage_v, acc_sh.at[pl.ds(sid * ROWS_PER_TILE + k * 128, 128)])
        plsc.subcore_barrier()

        sems = (sem0, sem1)

        def run_pass(table_hbm):
            pltpu.async_copy(table_hbm.at[src_v.at[0]], rows_v.at[0], sem0)
            pltpu.async_copy(table_hbm.at[src_v.at[1]], rows_v.at[1], sem1)

            def body(jh, carry):
                for b in range(2):
                    jj = jh * 2 + b
                    pltpu.make_async_copy(
                        table_hbm.at[src_v.at[jj]], rows_v.at[b],
                        sems[b]).wait()
                    pltpu.sync_copy(rows_v.at[b], acc_sh.at[dst_v.at[jj]],
                                    add=True)
                    pltpu.async_copy(
                        table_hbm.at[src_v.at[jj + 2]], rows_v.at[b], sems[b])
                return carry

            lax.fori_loop(0, CH // 2, body, 0)
            pltpu.make_async_copy(
                table_hbm.at[src_v.at[CH]], rows_v.at[0], sem0).wait()
            pltpu.make_async_copy(
                table_hbm.at[src_v.at[CH + 1]], rows_v.at[1], sem1).wait()

        @pl.when(cid == 0)
        def _():
            run_pass(ta_hbm)

        @pl.when(cid == 1)
        def _():
            run_pass(tb_hbm)

        plsc.subcore_barrier()

        for k in range(ROWS_PER_TILE // 128):
            r0 = sid * ROWS_PER_TILE + k * 128
            pltpu.sync_copy(acc_sh.at[pl.ds(r0, 128)], stage_v)
            pltpu.sync_copy(stage_v, out_hbm.at[pl.ds(cid * N_PAD + r0, 128)])

    return agg


_sc_agg = _make_sc_agg()


_DOT = functools.partial(
    jnp.dot, preferred_element_type=jnp.float32, precision=lax.Precision.DEFAULT)

BLK = 1024
GRID = N_PAD // BLK


def _neigh(aa_ref, ab_ref, rdeg, wn_ref):
    return (_DOT(aa_ref[:, 0:H] * rdeg, wn_ref[0:H, :])
            + _DOT(ab_ref[:, 0:H] * rdeg, wn_ref[H:D, :]))


def _accum_stats(z, z_ref, s1_ref, s2_ref):
    z_ref[...] = z

    @pl.when(pl.program_id(0) == 0)
    def _():
        s1_ref[...] = jnp.zeros_like(s1_ref)
        s2_ref[...] = jnp.zeros_like(s2_ref)

    row = (pl.program_id(0) * BLK
           + lax.broadcasted_iota(jnp.int32, (BLK, 1), 0))
    zm = jnp.where(row < N, z, 0.0)
    s1_ref[...] += jnp.sum(zm, axis=0, keepdims=True)
    s2_ref[...] += jnp.sum(zm * zm, axis=0, keepdims=True)


def _stats0_body(x_ref, aa_ref, ab_ref, ws_ref, wn_ref, b_ref,
                 z_ref, s1_ref, s2_ref, rdeg_ref):
    deg = jnp.sum(aa_ref[:, H:C], axis=1, keepdims=True)
    rdeg = 1.0 / jnp.maximum(deg, 1.0)
    rdeg_ref[...] = rdeg
    z = (_DOT(x_ref[...], ws_ref[...]) + _neigh(aa_ref, ab_ref, rdeg, wn_ref)
         + b_ref[...])
    _accum_stats(z, z_ref, s1_ref, s2_ref)


def _stats_mid_body(ha_ref, hb_ref, aa_ref, ab_ref, ws_ref, wn_ref, b_ref,
                    z_ref, s1_ref, s2_ref, rdeg_ref):
    rdeg = ha_ref[:, H:H + 1]
    rdeg_ref[...] = rdeg
    z = (_DOT(ha_ref[:, 0:H], ws_ref[0:H, :])
         + _DOT(hb_ref[:, 0:H], ws_ref[H:D, :])
         + _neigh(aa_ref, ab_ref, rdeg, wn_ref) + b_ref[...])
    _accum_stats(z, z_ref, s1_ref, s2_ref)


def _apply_body(z_ref, s1_ref, s2_ref, g_ref, be_ref, rdeg_ref,
                outa_ref, outb_ref):
    m = s1_ref[...] * (1.0 / N)
    v = s2_ref[...] * (1.0 / N) - m * m
    scale = g_ref[...] * lax.rsqrt(v + EPS)
    h = jnp.maximum(scale * (z_ref[...] - m) + be_ref[...], 0.0)
    rdeg = rdeg_ref[...]
    outa_ref[...] = jnp.concatenate(
        [h[:, 0:H], rdeg, jnp.zeros((BLK, C - H - 1), jnp.float32)], axis=1)
    outb_ref[...] = jnp.concatenate(
        [h[:, H:D], jnp.zeros((BLK, C - H), jnp.float32)], axis=1)


def _last_body(ha_ref, hb_ref, aa_ref, ab_ref, ws_ref, wn_ref, b_ref,
               out_ref):
    rdeg = ha_ref[:, H:H + 1]
    out_ref[...] = (_DOT(ha_ref[:, 0:H], ws_ref[0:H, :])
                    + _DOT(hb_ref[:, 0:H], ws_ref[H:D, :])
                    + _neigh(aa_ref, ab_ref, rdeg, wn_ref) + b_ref[...])


def _blk(i):
    return (i, 0)


def _blk1(i):
    return (i + GRID, 0)


def _fix(i):
    return (0, 0)


_HBLK = pl.BlockSpec((BLK, C), _blk)
_ROWBLK = pl.BlockSpec((BLK, D), _blk)
_RDEGBLK = pl.BlockSpec((BLK, 1), _blk)
_W_SPEC = pl.BlockSpec((D, D), _fix)
_ROW1 = pl.BlockSpec((1, D), _fix)
_S_SPEC = pl.BlockSpec((1, D), _fix)
_AGG_A = pl.BlockSpec((BLK, C), _blk)
_AGG_B = pl.BlockSpec((BLK, C), _blk1)

_Z_OUT = jax.ShapeDtypeStruct((N_PAD, D), jnp.float32)
_S_OUT = jax.ShapeDtypeStruct((1, D), jnp.float32)
_RDEG_OUT = jax.ShapeDtypeStruct((N_PAD, 1), jnp.float32)
_H_OUT = jax.ShapeDtypeStruct((N_PAD, C), jnp.float32)
_FINAL_OUT = jax.ShapeDtypeStruct((N, D), jnp.float32)

_stats0 = pl.pallas_call(
    _stats0_body,
    grid=(GRID,),
    in_specs=[_ROWBLK, _AGG_A, _AGG_B, _W_SPEC, _W_SPEC, _ROW1],
    out_specs=(_ROWBLK, _S_SPEC, _S_SPEC, _RDEGBLK),
    out_shape=(_Z_OUT, _S_OUT, _S_OUT, _RDEG_OUT))

_stats_mid = pl.pallas_call(
    _stats_mid_body,
    grid=(GRID,),
    in_specs=[_HBLK, _HBLK, _AGG_A, _AGG_B, _W_SPEC, _W_SPEC, _ROW1],
    out_specs=(_ROWBLK, _S_SPEC, _S_SPEC, _RDEGBLK),
    out_shape=(_Z_OUT, _S_OUT, _S_OUT, _RDEG_OUT))

_apply = pl.pallas_call(
    _apply_body,
    grid=(GRID,),
    in_specs=[_ROWBLK, _S_SPEC, _S_SPEC, _ROW1, _ROW1, _RDEGBLK],
    out_specs=(_HBLK, _HBLK),
    out_shape=(_H_OUT, _H_OUT))

_last = pl.pallas_call(
    _last_body,
    grid=(GRID,),
    in_specs=[_HBLK, _HBLK, _AGG_A, _AGG_B, _W_SPEC, _W_SPEC, _ROW1],
    out_specs=_ROWBLK,
    out_shape=_FINAL_OUT)


def kernel(x, edge_index, W_self_0, W_neigh_0, b_0, W_self_1, W_neigh_1, b_1,
           W_self_2, W_neigh_2, b_2, gamma_0, beta_0, gamma_1, beta_1):
    src = edge_index[0]
    dst = edge_index[1]

    pad = E_PAD - E
    ar = jnp.arange(pad, dtype=jnp.int32)
    src_p = jnp.concatenate([src, ar % N]).reshape(NT, CH, CHUNK)
    dst_p = jnp.concatenate([dst, N + ar % (N_PAD - N)])
    extra = (jnp.arange(NT * (CHP - CH) * CHUNK, dtype=jnp.int32) * 37) % N
    src_full = jnp.concatenate(
        [src_p, extra.reshape(NT, CHP - CH, CHUNK)], axis=1).reshape(
            NT * CHP, CHUNK)
    dst_full = dst_p.reshape(NT * CH, CHUNK)

    xa = jnp.pad(jnp.concatenate(
        [x[:, 0:H], jnp.ones((N, 1), jnp.float32),
         jnp.zeros((N, C - H - 1), jnp.float32)], axis=1),
        ((0, N_PAD - N), (0, 0)))
    xb = jnp.pad(jnp.concatenate(
        [x[:, H:D], jnp.zeros((N, C - H), jnp.float32)], axis=1),
        ((0, N_PAD - N), (0, 0)))

    b0 = b_0.reshape(1, D)
    b1 = b_1.reshape(1, D)
    b2 = b_2.reshape(1, D)
    g0 = gamma_0.reshape(1, D)
    g1 = gamma_1.reshape(1, D)
    be0 = beta_0.reshape(1, D)
    be1 = beta_1.reshape(1, D)

    agg0 = _sc_agg(xa, xb, src_full, dst_full)
    z0, s1_0, s2_0, rdeg = _stats0(x, agg0, agg0, W_self_0, W_neigh_0, b0)
    h1a, h1b = _apply(z0, s1_0, s2_0, g0, be0, rdeg)
    agg1 = _sc_agg(h1a, h1b, src_full, dst_full)
    z1, s1_1, s2_1, rdeg1 = _stats_mid(h1a, h1b, agg1, agg1,
                                       W_self_1, W_neigh_1, b1)
    h2a, h2b = _apply(z1, s1_1, s2_1, g1, be1, rdeg1)
    agg2 = _sc_agg(h2a, h2b, src_full, dst_full)
    return _last(h2a, h2b, agg2, agg2, W_self_2, W_neigh_2, b2)


# speedup vs baseline: 12.4215x; 1.4167x over previous
import functools

import jax
import jax.numpy as jnp
from jax import lax
from jax.experimental import pallas as pl
from jax.experimental.pallas import tpu as pltpu
from jax.experimental.pallas import tpu_sc as plsc

N = 10000
E = 320000
D = 128
C0 = 160
EPS = 1e-5

N_PAD = 10240
NW = 32
CHUNK = 128
CH = 80
E_PAD = NW * CH * CHUNK
CHP = 88
ROWS_PER_TILE = N_PAD // 16


def _make_sc_agg(width):
    mesh = plsc.VectorSubcoreMesh(core_axis_name="c", subcore_axis_name="s")

    @functools.partial(
        pl.kernel,
        out_type=jax.ShapeDtypeStruct((2 * N_PAD, width), jnp.bfloat16),
        mesh=mesh,
        scratch_types=[
            pltpu.VMEM((CHP, CHUNK), jnp.int32),
            pltpu.VMEM((CH, CHUNK), jnp.int32),
            pltpu.VMEM((2, CHUNK, width), jnp.bfloat16),
            pltpu.VMEM((128, width), jnp.bfloat16),
            pltpu.VMEM_SHARED((N_PAD, width), jnp.bfloat16),
            pltpu.SemaphoreType.DMA,
            pltpu.SemaphoreType.DMA,
        ],
        compiler_params=pltpu.CompilerParams(use_tc_tiling_on_sc=False),
    )
    def agg(table_hbm, src_hbm, dst_hbm, out_hbm,
            src_v, dst_v, rows_v, stage_v, acc_sh, sem0, sem1):
        cid = lax.axis_index("c")
        sid = lax.axis_index("s")
        w = cid * 16 + sid

        pltpu.sync_copy(src_hbm.at[pl.ds(w * CHP, CHP)], src_v)
        pltpu.sync_copy(dst_hbm.at[pl.ds(w * CH, CH)], dst_v)

        zv = jnp.zeros((32,), jnp.bfloat16)

        def zrow(i, carry):
            for j in range(width // 32):
                stage_v[i, pl.ds(j * 32, 32)] = zv
            return carry

        lax.fori_loop(0, 128, zrow, 0)
        for k in range(ROWS_PER_TILE // 128):
            pltpu.sync_copy(
                stage_v, acc_sh.at[pl.ds(sid * ROWS_PER_TILE + k * 128, 128)])
        plsc.subcore_barrier()

        sems = (sem0, sem1)
        pltpu.async_copy(table_hbm.at[src_v.at[0]], rows_v.at[0], sem0)
        pltpu.async_copy(table_hbm.at[src_v.at[1]], rows_v.at[1], sem1)

        def body(jh, carry):
            for b in range(2):
                jj = jh * 2 + b
                pltpu.make_async_copy(
                    table_hbm.at[src_v.at[jj]], rows_v.at[b], sems[b]).wait()
                pltpu.sync_copy(rows_v.at[b], acc_sh.at[dst_v.at[jj]], add=True)
                pltpu.async_copy(
                    table_hbm.at[src_v.at[jj + 2]], rows_v.at[b], sems[b])
            return carry

        lax.fori_loop(0, CH // 2, body, 0)
        pltpu.make_async_copy(
            table_hbm.at[src_v.at[CH]], rows_v.at[0], sem0).wait()
        pltpu.make_async_copy(
            table_hbm.at[src_v.at[CH + 1]], rows_v.at[1], sem1).wait()
        plsc.subcore_barrier()

        for k in range(ROWS_PER_TILE // 128):
            r0 = sid * ROWS_PER_TILE + k * 128
            pltpu.sync_copy(acc_sh.at[pl.ds(r0, 128)], stage_v)
            pltpu.sync_copy(stage_v, out_hbm.at[pl.ds(cid * N_PAD + r0, 128)])

    return agg


_agg0 = _make_sc_agg(C0)
_agg = _make_sc_agg(D)


_DOT = functools.partial(
    jnp.dot, preferred_element_type=jnp.float32, precision=lax.Precision.DEFAULT)

BLK = 1024
GRID = N_PAD // BLK


def _accum_stats(z, z_ref, s1_ref, s2_ref):
    z_ref[...] = z

    @pl.when(pl.program_id(0) == 0)
    def _():
        s1_ref[...] = jnp.zeros_like(s1_ref)
        s2_ref[...] = jnp.zeros_like(s2_ref)

    row = (pl.program_id(0) * BLK
           + lax.broadcasted_iota(jnp.int32, (BLK, 1), 0))
    zm = jnp.where(row < N, z, 0.0)
    s1_ref[...] += jnp.sum(zm, axis=0, keepdims=True)
    s2_ref[...] += jnp.sum(zm * zm, axis=0, keepdims=True)


def _stats0_body(x_ref, a0_ref, a1_ref, ws_ref, wn_ref, b_ref,
                 z_ref, s1_ref, s2_ref, rdeg_ref):
    a0 = a0_ref[...].astype(jnp.float32)
    a1 = a1_ref[...].astype(jnp.float32)
    deg = jnp.sum(a0[:, D:C0] + a1[:, D:C0], axis=1, keepdims=True)
    rdeg = 1.0 / jnp.maximum(deg, 1.0)
    rdeg_ref[...] = rdeg
    hn = (a0[:, 0:D] + a1[:, 0:D]) * rdeg
    z = (_DOT(x_ref[...], ws_ref[...]) + _DOT(hn, wn_ref[...]) + b_ref[...])
    _accum_stats(z, z_ref, s1_ref, s2_ref)


def _stats_mid_body(h_ref, rdeg_ref, a0_ref, a1_ref, ws_ref, wn_ref, b_ref,
                    z_ref, s1_ref, s2_ref):
    rdeg = rdeg_ref[...]
    hn = (a0_ref[...].astype(jnp.float32)
          + a1_ref[...].astype(jnp.float32)) * rdeg
    z = (_DOT(h_ref[...], ws_ref[...]) + _DOT(hn, wn_ref[...]) + b_ref[...])
    _accum_stats(z, z_ref, s1_ref, s2_ref)


def _apply_body(z_ref, s1_ref, s2_ref, g_ref, be_ref, out_ref, tab_ref):
    m = s1_ref[...] * (1.0 / N)
    v = s2_ref[...] * (1.0 / N) - m * m
    scale = g_ref[...] * lax.rsqrt(v + EPS)
    h = jnp.maximum(scale * (z_ref[...] - m) + be_ref[...], 0.0)
    out_ref[...] = h
    tab_ref[...] = h.astype(jnp.bfloat16)


def _last_body(h_ref, rdeg_ref, a0_ref, a1_ref, ws_ref, wn_ref, b_ref,
               out_ref):
    rdeg = rdeg_ref[...]
    hn = (a0_ref[...].astype(jnp.float32)
          + a1_ref[...].astype(jnp.float32)) * rdeg
    out_ref[...] = (_DOT(h_ref[...], ws_ref[...]) + _DOT(hn, wn_ref[...])
                    + b_ref[...])


def _blk(i):
    return (i, 0)


def _blk1(i):
    return (i + GRID, 0)


def _fix(i):
    return (0, 0)


_ROWBLK = pl.BlockSpec((BLK, D), _blk)
_RDEGBLK = pl.BlockSpec((BLK, 1), _blk)
_W_SPEC = pl.BlockSpec((D, D), _fix)
_ROW1 = pl.BlockSpec((1, D), _fix)
_S_SPEC = pl.BlockSpec((1, D), _fix)
_AGG0_A = pl.BlockSpec((BLK, C0), _blk)
_AGG0_B = pl.BlockSpec((BLK, C0), _blk1)
_AGG_A = pl.BlockSpec((BLK, D), _blk)
_AGG_B = pl.BlockSpec((BLK, D), _blk1)

_Z_OUT = jax.ShapeDtypeStruct((N_PAD, D), jnp.float32)
_S_OUT = jax.ShapeDtypeStruct((1, D), jnp.float32)
_RDEG_OUT = jax.ShapeDtypeStruct((N_PAD, 1), jnp.float32)
_H_OUT = jax.ShapeDtypeStruct((N_PAD, D), jnp.float32)
_TAB_OUT = jax.ShapeDtypeStruct((N_PAD, D), jnp.bfloat16)
_FINAL_OUT = jax.ShapeDtypeStruct((N, D), jnp.float32)

_stats0 = pl.pallas_call(
    _stats0_body,
    grid=(GRID,),
    in_specs=[_ROWBLK, _AGG0_A, _AGG0_B, _W_SPEC, _W_SPEC, _ROW1],
    out_specs=(_ROWBLK, _S_SPEC, _S_SPEC, _RDEGBLK),
    out_shape=(_Z_OUT, _S_OUT, _S_OUT, _RDEG_OUT))

_stats_mid = pl.pallas_call(
    _stats_mid_body,
    grid=(GRID,),
    in_specs=[_ROWBLK, _RDEGBLK, _AGG_A, _AGG_B, _W_SPEC, _W_SPEC, _ROW1],
    out_specs=(_ROWBLK, _S_SPEC, _S_SPEC),
    out_shape=(_Z_OUT, _S_OUT, _S_OUT))

_apply = pl.pallas_call(
    _apply_body,
    grid=(GRID,),
    in_specs=[_ROWBLK, _S_SPEC, _S_SPEC, _ROW1, _ROW1],
    out_specs=(_ROWBLK, _ROWBLK),
    out_shape=(_H_OUT, _TAB_OUT))

_last = pl.pallas_call(
    _last_body,
    grid=(GRID,),
    in_specs=[_ROWBLK, _RDEGBLK, _AGG_A, _AGG_B, _W_SPEC, _W_SPEC, _ROW1],
    out_specs=_ROWBLK,
    out_shape=_FINAL_OUT)


def kernel(x, edge_index, W_self_0, W_neigh_0, b_0, W_self_1, W_neigh_1, b_1,
           W_self_2, W_neigh_2, b_2, gamma_0, beta_0, gamma_1, beta_1):
    src = edge_index[0]
    dst = edge_index[1]

    pad = E_PAD - E
    ar = jnp.arange(pad, dtype=jnp.int32)
    src_p = jnp.concatenate([src, ar % N]).reshape(NW, CH, CHUNK)
    dst_p = jnp.concatenate([dst, N + ar % (N_PAD - N)])
    extra = (jnp.arange(NW * (CHP - CH) * CHUNK, dtype=jnp.int32) * 37) % N
    src_full = jnp.concatenate(
        [src_p, extra.reshape(NW, CHP - CH, CHUNK)], axis=1).reshape(
            NW * CHP, CHUNK)
    dst_full = dst_p.reshape(NW * CH, CHUNK)

    x0 = jnp.pad(jnp.concatenate(
        [x.astype(jnp.bfloat16),
         jnp.ones((N, 1), jnp.bfloat16),
         jnp.zeros((N, C0 - D - 1), jnp.bfloat16)], axis=1),
        ((0, N_PAD - N), (0, 0)))

    b0 = b_0.reshape(1, D)
    b1 = b_1.reshape(1, D)
    b2 = b_2.reshape(1, D)
    g0 = gamma_0.reshape(1, D)
    g1 = gamma_1.reshape(1, D)
    be0 = beta_0.reshape(1, D)
    be1 = beta_1.reshape(1, D)

    agg0 = _agg0(x0, src_full, dst_full)
    z0, s1_0, s2_0, rdeg = _stats0(x, agg0, agg0, W_self_0, W_neigh_0, b0)
    h1, t1 = _apply(z0, s1_0, s2_0, g0, be0)
    agg1 = _agg(t1, src_full, dst_full)
    z1, s1_1, s2_1 = _stats_mid(h1, rdeg, agg1, agg1, W_self_1, W_neigh_1, b1)
    h2, t2 = _apply(z1, s1_1, s2_1, g1, be1)
    agg2 = _agg(t2, src_full, dst_full)
    return _last(h2, rdeg, agg2, agg2, W_self_2, W_neigh_2, b2)
